# baseline (speedup 1.0000x reference)
import jax
import jax.numpy as jnp
from jax import lax
from jax.experimental import pallas as pl
from jax.experimental.pallas import tpu as pltpu
from jax.experimental.pallas import tpu_sc as plsc

VOCAB = 1000000
EMB = 64
B = 4096
L = 200

NC = 2
NS = 16
NW = NC * NS

ROWS = B * L
IW = 128
NIDX = ROWS // IW
G = 4
PER_W = NIDX // NW
ITERS = PER_W // G


def _body(idx_hbm, table_hbm, out_hbm, idx_v, rows_v, sem):
    wid = lax.axis_index("s") * NC + lax.axis_index("c")
    base = wid * PER_W

    def step(it, carry):
        row0 = base + it * G
        pltpu.sync_copy(idx_hbm.at[pl.ds(row0, G)], idx_v)
        handles = [
            pltpu.async_copy(table_hbm.at[idx_v.at[j]], rows_v.at[j], sem)
            for j in range(G)
        ]
        for h in handles:
            h.wait()
        pltpu.sync_copy(rows_v, out_hbm.at[pl.ds(row0, G)])
        return carry

    lax.fori_loop(0, ITERS, step, 0)


@jax.jit
def kernel(idx, table):
    idx2 = idx.reshape(NIDX, IW).astype(jnp.int32)
    mesh = plsc.VectorSubcoreMesh(
        core_axis_name="c", subcore_axis_name="s", num_cores=NC, num_subcores=NS
    )
    out = pl.kernel(
        _body,
        out_type=jax.ShapeDtypeStruct((NIDX, IW, EMB), jnp.float32),
        mesh=mesh,
        scratch_types=[
            pltpu.VMEM((G, IW), jnp.int32),
            pltpu.VMEM((G, IW, EMB), jnp.float32),
            pltpu.SemaphoreType.DMA,
        ],
        compiler_params=pltpu.CompilerParams(use_tc_tiling_on_sc=False),
    )(idx2, table)
    return out.reshape(B, L, EMB)


# speedup vs baseline: 1.0469x; 1.0469x over previous
import jax
import jax.numpy as jnp
from jax import lax
from jax.experimental import pallas as pl
from jax.experimental.pallas import tpu as pltpu
from jax.experimental.pallas import tpu_sc as plsc

VOCAB = 1000000
EMB = 64
B = 4096
L = 200

NC = 2
NS = 16
NW = NC * NS

ROWS = B * L
IW = 128
NIDX = ROWS // IW
PER_W = NIDX // NW
NBUF = 8
GLAG = 4
OUTER = PER_W // NBUF


def _body(idx_hbm, table_hbm, out_hbm, idx_v, rows_v, gsem, ssem):
    wid = lax.axis_index("s") * NC + lax.axis_index("c")
    base = wid * PER_W
    pltpu.sync_copy(idx_hbm.at[pl.ds(base, PER_W)], idx_v)

    def fire_gather(i, b):
        return pltpu.async_copy(
            table_hbm.at[idx_v.at[i]], rows_v.at[b], gsem.at[b]
        )

    def wait_gather(i, b):
        pltpu.make_async_copy(
            table_hbm.at[idx_v.at[i]], rows_v.at[b], gsem.at[b]
        ).wait()

    def fire_store(i, b):
        return pltpu.async_copy(rows_v.at[b], out_hbm.at[base + i], ssem.at[b])

    def wait_store(i, b):
        pltpu.make_async_copy(
            rows_v.at[b], out_hbm.at[base + i], ssem.at[b]
        ).wait()

    for b in range(NBUF):
        fire_gather(b, b)
    for j in range(NBUF - GLAG):
        wait_gather(j, j)
        fire_store(j, j)

    def outer(o, carry):
        for b in range(NBUF):
            i = o * NBUF + b
            wait_store(i - NBUF, b)
            fire_gather(i, b)
            j = i - GLAG
            bj = (b + NBUF - GLAG) % NBUF
            wait_gather(j, bj)
            fire_store(j, bj)
        return carry

    lax.fori_loop(1, OUTER, outer, 0)

    last = OUTER * NBUF
    for k in range(GLAG):
        i = last - GLAG + k
        b = i % NBUF
        wait_gather(i, b)
        fire_store(i, b)
    for k in range(NBUF):
        i = last - NBUF + k
        wait_store(i, i % NBUF)


@jax.jit
def kernel(idx, table):
    idx2 = idx.reshape(NIDX, IW).astype(jnp.int32)
    mesh = plsc.VectorSubcoreMesh(
        core_axis_name="c", subcore_axis_name="s", num_cores=NC, num_subcores=NS
    )
    out = pl.kernel(
        _body,
        out_type=jax.ShapeDtypeStruct((NIDX, IW, EMB), jnp.float32),
        mesh=mesh,
        scratch_types=[
            pltpu.VMEM((PER_W, IW), jnp.int32),
            pltpu.VMEM((NBUF, IW, EMB), jnp.float32),
            pltpu.SemaphoreType.DMA((NBUF,)),
            pltpu.SemaphoreType.DMA((NBUF,)),
        ],
        compiler_params=pltpu.CompilerParams(use_tc_tiling_on_sc=False),
    )(idx2, table)
    return out.reshape(B, L, EMB)
